# Initial kernel scaffold; baseline (speedup 1.0000x reference)
import jax
import jax.numpy as jnp
from jax.experimental import pallas as pl


def kernel(x, edge_attr, edge_index, Wf0, bf0, Ws0, bs0, Wf1, bf1, Ws1, bs1, Wf2, bf2, Ws2, bs2, bn_gamma, bn_beta, bn_mean, bn_var, Wc, bc):
    raise NotImplementedError("write your pallas kernel here")


# speedup vs baseline: 2.7368x; 2.7368x over previous
import functools

import jax
import jax.numpy as jnp
from jax import lax
from jax.experimental import pallas as pl
from jax.experimental.pallas import tpu as pltpu, tpu_sc as plsc

N = 10000
E = 320000
D = 128
DE = 16
NC = 40

NCORE = 2
NSUB = 16
NW = NCORE * NSUB
EPW = E // NW
C = 40
NCHUNK = EPW // C
ROWS_PER_SUB = 624

_sc_mesh = plsc.VectorSubcoreMesh(core_axis_name="c", subcore_axis_name="s")


def _gather_body(h_hbm, dst_hbm, src_hbm, out_hbm,
                 dst_v, src_v, pd0, pd1, ps0, ps1,
                 gd0, gd1, gs0, gs1, wd0, wd1, ws0, ws1):
    cid = lax.axis_index("c")
    sid = lax.axis_index("s")
    wid = cid * NSUB + sid
    wbase = wid * EPW

    pltpu.sync_copy(dst_hbm.at[wid], dst_v)
    pltpu.sync_copy(src_hbm.at[wid], src_v)

    pds = (pd0, pd1)
    pss = (ps0, ps1)
    gds = (gd0, gd1)
    gss = (gs0, gs1)
    wds = (wd0, wd1)
    wss = (ws0, ws1)

    def start_gather(c, b):
        pltpu.async_copy(h_hbm.at[dst_v.at[c]], pds[b], gds[b])
        pltpu.async_copy(h_hbm.at[src_v.at[c]], pss[b], gss[b])

    start_gather(0, 0)
    start_gather(1, 1)

    def body(k, _):
        for b in range(2):
            c = 2 * k + b
            pltpu.make_async_copy(h_hbm.at[dst_v.at[c]], pds[b], gds[b]).wait()
            pltpu.make_async_copy(h_hbm.at[src_v.at[c]], pss[b], gss[b]).wait()

            pltpu.async_copy(pds[b], out_hbm.at[0, pl.ds(wbase + c * C, C)], wds[b])
            pltpu.async_copy(pss[b], out_hbm.at[1, pl.ds(wbase + c * C, C)], wss[b])

            @pl.when(c + 2 < NCHUNK)
            def _():
                pltpu.make_async_copy(pds[b], out_hbm.at[0, pl.ds(wbase, C)], wds[b]).wait()
                pltpu.make_async_copy(pss[b], out_hbm.at[1, pl.ds(wbase, C)], wss[b]).wait()
                start_gather(c + 2, b)
        return 0

    lax.fori_loop(0, NCHUNK // 2, body, 0, unroll=False)

    for b in range(2):
        pltpu.make_async_copy(pds[b], out_hbm.at[0, pl.ds(wbase, C)], wds[b]).wait()
        pltpu.make_async_copy(pss[b], out_hbm.at[1, pl.ds(wbase, C)], wss[b]).wait()


@functools.partial(
    pl.kernel,
    out_type=jax.ShapeDtypeStruct((2, E, D), jnp.float32),
    mesh=_sc_mesh,
    scratch_types=[
        pltpu.VMEM((NCHUNK, C), jnp.int32),
        pltpu.VMEM((NCHUNK, C), jnp.int32),
        pltpu.VMEM((C, D), jnp.float32),
        pltpu.VMEM((C, D), jnp.float32),
        pltpu.VMEM((C, D), jnp.float32),
        pltpu.VMEM((C, D), jnp.float32),
        pltpu.SemaphoreType.DMA,
        pltpu.SemaphoreType.DMA,
        pltpu.SemaphoreType.DMA,
        pltpu.SemaphoreType.DMA,
        pltpu.SemaphoreType.DMA,
        pltpu.SemaphoreType.DMA,
        pltpu.SemaphoreType.DMA,
        pltpu.SemaphoreType.DMA,
    ],
)
def _sc_gather(h_hbm, dst_hbm, src_hbm, out_hbm, *rest):
    _gather_body(h_hbm, dst_hbm, src_hbm, out_hbm, *rest)


def _scatter_body(msg_hbm, dst_hbm, out_hbm, dst_v, mb0, mb1, zbuf, agg,
                  m0, m1):
    cid = lax.axis_index("c")
    sid = lax.axis_index("s")
    wid = cid * NSUB + sid

    pltpu.sync_copy(dst_hbm.at[wid], dst_v)

    for g in range(8):
        sl = pl.ds(g * 16, 16)
        for r in range(16):
            zbuf[r, sl] = jnp.zeros((16,), jnp.float32)
    nz = jnp.where(sid == NSUB - 1, 40, 39)
    zbase = sid * ROWS_PER_SUB

    def zrow(k, _):
        pltpu.sync_copy(zbuf, agg.at[pl.ds(zbase + k * 16, 16)])
        return 0

    lax.fori_loop(0, nz, zrow, 0, unroll=False)
    plsc.subcore_barrier()

    mbs = (mb0, mb1)
    msems = (m0, m1)

    def start_load(c, b):
        pltpu.async_copy(msg_hbm.at[wid, c], mbs[b], msems[b])

    start_load(0, 0)
    start_load(1, 1)

    def body(k, _):
        for b in range(2):
            c = 2 * k + b
            pltpu.make_async_copy(msg_hbm.at[wid, c], mbs[b], msems[b]).wait()
            pltpu.sync_copy(mbs[b], agg.at[dst_v.at[c]], add=True)

            @pl.when(c + 2 < NCHUNK)
            def _():
                start_load(c + 2, b)
        return 0

    lax.fori_loop(0, NCHUNK // 2, body, 0, unroll=False)
    plsc.subcore_barrier()

    pltpu.sync_copy(agg.at[pl.ds(zbase, ROWS_PER_SUB)],
                    out_hbm.at[cid, pl.ds(zbase, ROWS_PER_SUB)])

    @pl.when(sid == NSUB - 1)
    def _():
        pltpu.sync_copy(agg.at[pl.ds(NSUB * ROWS_PER_SUB, N - NSUB * ROWS_PER_SUB)],
                        out_hbm.at[cid, pl.ds(NSUB * ROWS_PER_SUB, N - NSUB * ROWS_PER_SUB)])


@functools.partial(
    pl.kernel,
    out_type=jax.ShapeDtypeStruct((NCORE, N, D), jnp.float32),
    mesh=_sc_mesh,
    scratch_types=[
        pltpu.VMEM((NCHUNK, C), jnp.int32),
        pltpu.VMEM((C, D), jnp.float32),
        pltpu.VMEM((C, D), jnp.float32),
        pltpu.VMEM((16, D), jnp.float32),
        pltpu.VMEM_SHARED((N, D), jnp.float32),
        pltpu.SemaphoreType.DMA,
        pltpu.SemaphoreType.DMA,
    ],
)
def _sc_scatter(msg_hbm, dst_hbm, out_hbm, *rest):
    _scatter_body(msg_hbm, dst_hbm, out_hbm, *rest)


_NB = 10
_RB = N // _NB
_EB = 4000


def _msg_body(hd_ref, hs_ref, ea_ref, wf_ref, bf_ref, ws_ref, bs_ref, out_ref):
    z = jnp.concatenate([hd_ref[...], hs_ref[...], ea_ref[...]], axis=1)
    zf = jnp.dot(z, wf_ref[...], preferred_element_type=jnp.float32) + bf_ref[...]
    zs = jnp.dot(z, ws_ref[...], preferred_element_type=jnp.float32) + bs_ref[...]
    sig = 1.0 / (1.0 + jnp.exp(-zf))
    sp = jnp.maximum(zs, 0.0) + jnp.log1p(jnp.exp(-jnp.abs(zs)))
    out_ref[...] = sig * sp


def _msg(hd, hs, ea, wf, bf, ws, bs):
    return pl.pallas_call(
        _msg_body,
        grid=(E // _EB,),
        in_specs=[
            pl.BlockSpec((_EB, D), lambda i: (i, 0)),
            pl.BlockSpec((_EB, D), lambda i: (i, 0)),
            pl.BlockSpec((_EB, DE), lambda i: (i, 0)),
            pl.BlockSpec((2 * D + DE, D), lambda i: (0, 0)),
            pl.BlockSpec((1, D), lambda i: (0, 0)),
            pl.BlockSpec((2 * D + DE, D), lambda i: (0, 0)),
            pl.BlockSpec((1, D), lambda i: (0, 0)),
        ],
        out_specs=pl.BlockSpec((_EB, D), lambda i: (i, 0)),
        out_shape=jax.ShapeDtypeStruct((E, D), jnp.float32),
    )(hd, hs, ea, wf, bf, ws, bs)


def _upd_body(h_ref, a0_ref, a1_ref, out_ref):
    out_ref[...] = jnp.maximum(h_ref[...] + a0_ref[...] + a1_ref[...], 0.0)


def _upd(h, a0, a1):
    return pl.pallas_call(
        _upd_body,
        grid=(_NB,),
        in_specs=[
            pl.BlockSpec((_RB, D), lambda i: (i, 0)),
            pl.BlockSpec((_RB, D), lambda i: (i, 0)),
            pl.BlockSpec((_RB, D), lambda i: (i, 0)),
        ],
        out_specs=pl.BlockSpec((_RB, D), lambda i: (i, 0)),
        out_shape=jax.ShapeDtypeStruct((N, D), jnp.float32),
    )(h, a0, a1)


def _head_body(h_ref, a0_ref, a1_ref, mean_ref, var_ref, gam_ref, bet_ref,
               wc_ref, bc_ref, am_ref, emb_ref, pred_ref):
    h3 = h_ref[...] + a0_ref[...] + a1_ref[...]
    emb = (h3 - mean_ref[...]) / jnp.sqrt(var_ref[...] + 1e-5) * gam_ref[...] + bet_ref[...]
    emb_ref[...] = emb
    logits = jnp.dot(emb, wc_ref[...], preferred_element_type=jnp.float32) + bc_ref[...]
    m = jnp.max(logits, axis=1, keepdims=True)
    ex = jnp.exp(logits - m)
    pred = ex / jnp.sum(ex, axis=1, keepdims=True)
    pred_ref[...] = pred
    pm = jnp.max(pred, axis=1, keepdims=True)
    ids = lax.broadcasted_iota(jnp.int32, pred.shape, 1)
    am_ref[...] = jnp.min(jnp.where(pred == pm, ids, NC), axis=1, keepdims=True)


def _head(h2, a0, a1, mean, var, gam, bet, wc, bc):
    return pl.pallas_call(
        _head_body,
        grid=(_NB,),
        in_specs=[
            pl.BlockSpec((_RB, D), lambda i: (i, 0)),
            pl.BlockSpec((_RB, D), lambda i: (i, 0)),
            pl.BlockSpec((_RB, D), lambda i: (i, 0)),
            pl.BlockSpec((1, D), lambda i: (0, 0)),
            pl.BlockSpec((1, D), lambda i: (0, 0)),
            pl.BlockSpec((1, D), lambda i: (0, 0)),
            pl.BlockSpec((1, D), lambda i: (0, 0)),
            pl.BlockSpec((D, NC), lambda i: (0, 0)),
            pl.BlockSpec((1, NC), lambda i: (0, 0)),
        ],
        out_specs=[
            pl.BlockSpec((_RB, 1), lambda i: (i, 0)),
            pl.BlockSpec((_RB, D), lambda i: (i, 0)),
            pl.BlockSpec((_RB, NC), lambda i: (i, 0)),
        ],
        out_shape=[
            jax.ShapeDtypeStruct((N, 1), jnp.int32),
            jax.ShapeDtypeStruct((N, D), jnp.float32),
            jax.ShapeDtypeStruct((N, NC), jnp.float32),
        ],
    )(h2, a0, a1, mean, var, gam, bet, wc, bc)


def kernel(x, edge_attr, edge_index, Wf0, bf0, Ws0, bs0, Wf1, bf1, Ws1, bs1,
           Wf2, bf2, Ws2, bs2, bn_gamma, bn_beta, bn_mean, bn_var, Wc, bc):
    src = edge_index[0].astype(jnp.int32)
    dst = edge_index[1].astype(jnp.int32)
    dst3 = dst.reshape(NW, NCHUNK, C)
    src3 = src.reshape(NW, NCHUNK, C)

    def layer(h, Wf, bf, Ws, bs):
        g = _sc_gather(h, dst3, src3)
        msg = _msg(g[0], g[1], edge_attr, Wf, bf[None, :], Ws, bs[None, :])
        parts = _sc_scatter(msg.reshape(NW, NCHUNK, C, D), dst3)
        return parts[0], parts[1]

    a0, a1 = layer(x, Wf0, bf0, Ws0, bs0)
    h1 = _upd(x, a0, a1)
    a0, a1 = layer(h1, Wf1, bf1, Ws1, bs1)
    h2 = _upd(h1, a0, a1)
    a0, a1 = layer(h2, Wf2, bf2, Ws2, bs2)

    am, emb, pred = _head(h2, a0, a1, bn_mean[None, :], bn_var[None, :],
                          bn_gamma[None, :], bn_beta[None, :], Wc, bc[None, :])
    return (am.reshape(N), emb, pred)
